# baseline (speedup 1.0000x reference)
import functools

import jax
import jax.numpy as jnp
from jax import lax
from jax.experimental import pallas as pl
from jax.experimental.pallas import tpu as pltpu
from jax.experimental.pallas import tpu_sc as plsc

N = 10000
NP = 10240
NPB = NP // 8
E = 320000
ITERS = 7
NI = 9
NE = 11
NF = 10
NO = 9
MS = 96
FP = 16
G3 = 48

NC = 2
NS = 16
NW = NC * NS
EPW = E // NW
SW = 80
SPB = 5
BLK = SW * SPB
NBLK = EPW // BLK
IDXR = EPW // SW
RPT = NP // NS

EB = E // 8
WB = EPW // 8
BB = BLK // 8
RB = 800
MS8 = 8 * MS
_f32 = jnp.float32


def _mesh():
    return plsc.VectorSubcoreMesh(core_axis_name="c", subcore_axis_name="s",
                                  num_cores=NC, num_subcores=NS)


_SC_PARAMS = pltpu.CompilerParams(use_tc_tiling_on_sc=False)


def _gather_body(h_hbm, srcx, dstx, xs_hbm, xd_hbm,
                 idx_s, idx_d, rows_s, rows_d, stage, h_sh, sem_s, sem_d):
    c = lax.axis_index("c")
    s = lax.axis_index("s")
    wid = s * NC + c
    pltpu.sync_copy(h_hbm.at[pl.ds(s * RPT, RPT)], stage)
    pltpu.sync_copy(stage, h_sh.at[pl.ds(s * RPT, RPT)])
    pltpu.sync_copy(srcx.at[wid], idx_s)
    pltpu.sync_copy(dstx.at[wid], idx_d)
    plsc.subcore_barrier()

    def blk(b, carry):
        cps = []
        for t in range(SPB):
            r = b * SPB + t
            cps.append(pltpu.async_copy(h_sh.at[idx_s.at[r]],
                                        rows_s.at[pl.ds(t * SW, SW)], sem_s))
            cps.append(pltpu.async_copy(h_sh.at[idx_d.at[r]],
                                        rows_d.at[pl.ds(t * SW, SW)], sem_d))
        for cp in cps:
            cp.wait()
        off = wid * EPW + b * BLK
        pltpu.sync_copy(rows_s, xs_hbm.at[pl.ds(off, BLK)])
        pltpu.sync_copy(rows_d, xd_hbm.at[pl.ds(off, BLK)])
        return carry

    lax.fori_loop(0, NBLK, blk, 0)


def _gather_call(h, src2, dst2):
    out_type = (jax.ShapeDtypeStruct((E, FP), _f32),
                jax.ShapeDtypeStruct((E, FP), _f32))
    return pl.kernel(
        _gather_body,
        out_type=out_type,
        mesh=_mesh(),
        scratch_types=[
            pltpu.VMEM((IDXR, SW), jnp.int32),
            pltpu.VMEM((IDXR, SW), jnp.int32),
            pltpu.VMEM((BLK, FP), _f32),
            pltpu.VMEM((BLK, FP), _f32),
            pltpu.VMEM((RPT, FP), _f32),
            pltpu.VMEM_SHARED((NP, FP), _f32),
            pltpu.SemaphoreType.DMA,
            pltpu.SemaphoreType.DMA,
        ],
        compiler_params=_SC_PARAMS,
    )(h, src2, dst2)


def _scatter_body(msgs_hbm, dstx, aggp_hbm, idx_d, rows, flat, shared_agg, sem):
    c = lax.axis_index("c")
    s = lax.axis_index("s")
    wid = s * NC + c

    def zrow(i, carry):
        flat[i] = jnp.zeros((FP,), _f32)
        return carry

    lax.fori_loop(0, RPT, zrow, 0)
    pltpu.sync_copy(flat, shared_agg.at[pl.ds(s * RPT, RPT)])
    plsc.subcore_barrier()

    pltpu.sync_copy(dstx.at[wid], idx_d)

    def blk(b, carry):
        off = wid * EPW + b * BLK
        pltpu.sync_copy(msgs_hbm.at[pl.ds(off, BLK)], rows)
        cps = []
        for t in range(SPB):
            r = b * SPB + t
            cps.append(pltpu.async_copy(rows.at[pl.ds(t * SW, SW)],
                                        shared_agg.at[idx_d.at[r]], sem,
                                        add=True))
        for cp in cps:
            cp.wait()
        return carry

    lax.fori_loop(0, NBLK, blk, 0)
    plsc.subcore_barrier()
    pltpu.sync_copy(shared_agg.at[pl.ds(s * RPT, RPT)], flat)
    pltpu.sync_copy(flat, aggp_hbm.at[c, pl.ds(s * RPT, RPT)])


def _scatter_call(msgs, dst2):
    return pl.kernel(
        _scatter_body,
        out_type=jax.ShapeDtypeStruct((NC, NP, FP), _f32),
        mesh=_mesh(),
        scratch_types=[
            pltpu.VMEM((IDXR, SW), jnp.int32),
            pltpu.VMEM((BLK, FP), _f32),
            pltpu.VMEM((RPT, FP), _f32),
            pltpu.VMEM_SHARED((NP, FP), _f32),
            pltpu.SemaphoreType.DMA,
        ],
        compiler_params=_SC_PARAMS,
    )(msgs, dst2)


def _degree_body(dstx, degp_hbm, idx_d, ones, flat, shared_deg, sem):
    c = lax.axis_index("c")
    s = lax.axis_index("s")
    wid = s * NC + c

    def zrow(i, carry):
        flat[i] = jnp.zeros((FP,), _f32)
        return carry

    lax.fori_loop(0, RPT, zrow, 0)

    def orow(i, carry):
        ones[i] = jnp.ones((FP,), _f32)
        return carry

    lax.fori_loop(0, SW, orow, 0)
    pltpu.sync_copy(flat, shared_deg.at[pl.ds(s * RPT, RPT)])
    plsc.subcore_barrier()
    pltpu.sync_copy(dstx.at[wid], idx_d)

    def blk(b, carry):
        cps = []
        for t in range(SPB):
            r = b * SPB + t
            cps.append(pltpu.async_copy(ones, shared_deg.at[idx_d.at[r]], sem,
                                        add=True))
        for cp in cps:
            cp.wait()
        return carry

    lax.fori_loop(0, NBLK, blk, 0)
    plsc.subcore_barrier()
    pltpu.sync_copy(shared_deg.at[pl.ds(s * RPT, RPT)], flat)
    pltpu.sync_copy(flat, degp_hbm.at[c, pl.ds(s * RPT, RPT)])


def _degree_call(dst2):
    return pl.kernel(
        _degree_body,
        out_type=jax.ShapeDtypeStruct((NC, NP, FP), _f32),
        mesh=_mesh(),
        scratch_types=[
            pltpu.VMEM((IDXR, SW), jnp.int32),
            pltpu.VMEM((SW, FP), _f32),
            pltpu.VMEM((RPT, FP), _f32),
            pltpu.VMEM_SHARED((NP, FP), _f32),
            pltpu.SemaphoreType.DMA,
        ],
        compiler_params=_SC_PARAMS,
    )(dst2)


def _mlp_body(xs_ref, xd_ref, w1a_ref, w1b_ref, b1_ref, w2_ref, b2_ref,
              w3_ref, b3_ref, w4_ref, b4_ref, out_ref):
    h = (jnp.dot(xs_ref[...], w1a_ref[...], preferred_element_type=_f32)
         + jnp.dot(xd_ref[...], w1b_ref[...], preferred_element_type=_f32)
         + b1_ref[...])
    h = jnp.maximum(h, 0.0)
    h = jnp.dot(h, w2_ref[...], preferred_element_type=_f32) + b2_ref[...]
    h = jnp.maximum(h, 0.0)
    h = jnp.dot(h, w3_ref[...], preferred_element_type=_f32) + b3_ref[...]
    h = jnp.maximum(h, 0.0)
    out_ref[...] = (jnp.dot(h, w4_ref[...], preferred_element_type=_f32)
                    + b4_ref[...])


def _mlp_call(xs, xd, w1a, w1b, b1r, w2bd, b2r, w3bd, b3r, w4bd, b4r):
    def wspec(a):
        return pl.BlockSpec(a.shape, lambda i: (0,) * a.ndim)

    return pl.pallas_call(
        _mlp_body,
        grid=(EB // RB,),
        in_specs=[
            pl.BlockSpec((RB, 128), lambda i: (i, 0)),
            pl.BlockSpec((RB, 128), lambda i: (i, 0)),
            wspec(w1a), wspec(w1b), wspec(b1r), wspec(w2bd), wspec(b2r),
            wspec(w3bd), wspec(b3r), wspec(w4bd), wspec(b4r),
        ],
        out_specs=pl.BlockSpec((RB, 128), lambda i: (i, 0)),
        out_shape=jax.ShapeDtypeStruct((EB, 128), _f32),
    )(xs, xd, w1a, w1b, b1r, w2bd, b2r, w3bd, b3r, w4bd, b4r)


def _gru_body(aggp_ref, ni_ref, h_ref, wia_ref, wib_ref, bih_ref,
              whh_ref, bhh_ref, wf_ref, bf_ref, hout_ref, o_ref):
    agg = aggp_ref[0] + aggp_ref[1]
    gi = (jnp.dot(agg, wia_ref[...], preferred_element_type=_f32)
          + jnp.dot(ni_ref[...], wib_ref[...], preferred_element_type=_f32)
          + bih_ref[...])
    h = h_ref[...]
    gh = jnp.dot(h, whh_ref[...], preferred_element_type=_f32) + bhh_ref[...]
    r = jax.nn.sigmoid(gi[:, 0:128] + gh[:, 0:128])
    z = jax.nn.sigmoid(gi[:, 128:256] + gh[:, 128:256])
    n = jnp.tanh(gi[:, 256:384] + r * gh[:, 256:384])
    hn = (1.0 - z) * n + z * h
    hout_ref[...] = hn
    o_ref[...] = jnp.dot(hn, wf_ref[...], preferred_element_type=_f32) + bf_ref[...]


def _gru_call(aggp8, ni8, h8, wia8, wib8, bih8, whh8, bhh8, wf8, bf8):
    return pl.pallas_call(
        _gru_body,
        out_shape=(jax.ShapeDtypeStruct((NPB, 128), _f32),
                   jax.ShapeDtypeStruct((NPB, 128), _f32)),
    )(aggp8, ni8, h8, wia8, wib8, bih8, whh8, bhh8, wf8, bf8)


def _gru0_body(degp_ref, ni_ref, wia_ref, wib_ref, bih_ref, bhh_ref,
               wf_ref, bf_ref, b1_ref, w2_ref, b2_ref, w3_ref, b3_ref,
               w4_ref, b4_ref, hout_ref, o_ref):
    t = jnp.maximum(b1_ref[...], 0.0)
    t = jnp.maximum(
        jnp.dot(t, w2_ref[...], preferred_element_type=_f32) + b2_ref[...], 0.0)
    t = jnp.maximum(
        jnp.dot(t, w3_ref[...], preferred_element_type=_f32) + b3_ref[...], 0.0)
    m0 = jnp.dot(t, w4_ref[...], preferred_element_type=_f32) + b4_ref[...]
    agg = (degp_ref[0] + degp_ref[1]) * jnp.tile(m0, (1, 8))
    gi = (jnp.dot(agg, wia_ref[...], preferred_element_type=_f32)
          + jnp.dot(ni_ref[...], wib_ref[...], preferred_element_type=_f32)
          + bih_ref[...])
    gh = bhh_ref[...]
    r = jax.nn.sigmoid(gi[:, 0:128] + gh[:, 0:128])
    z = jax.nn.sigmoid(gi[:, 128:256] + gh[:, 128:256])
    n = jnp.tanh(gi[:, 256:384] + r * gh[:, 256:384])
    hn = (1.0 - z) * n
    hout_ref[...] = hn
    o_ref[...] = jnp.dot(hn, wf_ref[...], preferred_element_type=_f32) + bf_ref[...]


def _gru0_call(degp8, ni8, wia8, wib8, bih8, bhh8, wf8, bf8,
               b1o, w2o, b2o, w3o, b3o, w4o, b4o):
    return pl.pallas_call(
        _gru0_body,
        out_shape=(jax.ShapeDtypeStruct((NPB, 128), _f32),
                   jax.ShapeDtypeStruct((NPB, 128), _f32)),
    )(degp8, ni8, wia8, wib8, bih8, bhh8, wf8, bf8,
      b1o, w2o, b2o, w3o, b3o, w4o, b4o)


def kernel(node_inputs, src_ids, dst_ids, W1, b1, W2, b2, W3, b3, W4, b4,
           W_ih, b_ih, W_hh, b_hh, Wf, bf):
    src2 = src_ids.astype(jnp.int32).reshape(NW, IDXR, SW)
    dst2 = dst_ids.astype(jnp.int32).reshape(NW, IDXR, SW)
    ni_p = jnp.pad(node_inputs.astype(_f32), ((0, NP - N), (0, FP - NI)))

    w1a = jnp.zeros((128, MS8), _f32)
    w1b = jnp.zeros((128, MS8), _f32)
    w4bd = jnp.zeros((MS8, 128), _f32)
    b4r = jnp.zeros((1, 128), _f32)
    for e in range(8):
        w1a = w1a.at[16 * e:16 * e + NF, MS * e:MS * (e + 1)].set(W1[0:NF])
        w1b = w1b.at[16 * e:16 * e + NF, MS * e:MS * (e + 1)].set(W1[NF:2 * NF])
        w4bd = w4bd.at[MS * e:MS * (e + 1), 16 * e:16 * e + NE].set(W4)
        b4r = b4r.at[0, 16 * e:16 * e + NE].set(b4)
    eye8 = jnp.eye(8, dtype=_f32)
    w2bd = jnp.kron(eye8, W2)
    w3bd = jnp.kron(eye8, W3)
    b1r = jnp.tile(b1, 8)[None, :]
    b2r = jnp.tile(b2, 8)[None, :]
    b3r = jnp.tile(b3, 8)[None, :]

    wia8 = jnp.zeros((128, 384), _f32)
    wib8 = jnp.zeros((128, 384), _f32)
    whh8 = jnp.zeros((128, 384), _f32)
    bih8 = jnp.zeros((1, 384), _f32)
    bhh8 = jnp.zeros((1, 384), _f32)
    wf8 = jnp.zeros((128, 128), _f32)
    bf8 = jnp.zeros((1, 128), _f32)
    for e in range(8):
        for g in range(3):
            c0 = 128 * g + 16 * e
            wia8 = wia8.at[16 * e:16 * e + NE, c0:c0 + NF].set(
                W_ih[0:NE, NF * g:NF * (g + 1)])
            wib8 = wib8.at[16 * e:16 * e + NI, c0:c0 + NF].set(
                W_ih[NE:NE + NI, NF * g:NF * (g + 1)])
            whh8 = whh8.at[16 * e:16 * e + NF, c0:c0 + NF].set(
                W_hh[:, NF * g:NF * (g + 1)])
            bih8 = bih8.at[0, c0:c0 + NF].set(b_ih[NF * g:NF * (g + 1)])
            bhh8 = bhh8.at[0, c0:c0 + NF].set(b_hh[NF * g:NF * (g + 1)])
        wf8 = wf8.at[16 * e:16 * e + NF, 16 * e:16 * e + NO].set(Wf)
        bf8 = bf8.at[0, 16 * e:16 * e + NO].set(bf)
    ni8 = ni_p.reshape(NPB, 128)

    b1o = b1[None, :]
    b2o = b2[None, :]
    b3o = b3[None, :]
    w4o = jnp.zeros((MS, FP), _f32).at[:, :NE].set(W4)
    b4o = jnp.zeros((1, FP), _f32).at[0, :NE].set(b4)

    outs = []
    degp = _degree_call(dst2)
    h8, o8 = _gru0_call(degp.reshape(NC, NPB, 128), ni8, wia8, wib8, bih8,
                        bhh8, wf8, bf8, b1o, W2, b2o, W3, b3o, w4o, b4o)
    outs.append(o8.reshape(NP, FP)[:N, :NO])
    for _ in range(ITERS - 1):
        xs, xd = _gather_call(h8.reshape(NP, FP), src2, dst2)
        msgs8 = _mlp_call(xs.reshape(EB, 128), xd.reshape(EB, 128),
                          w1a, w1b, b1r, w2bd, b2r, w3bd, b3r, w4bd, b4r)
        aggp = _scatter_call(msgs8.reshape(E, FP), dst2)
        h8, o8 = _gru_call(aggp.reshape(NC, NPB, 128), ni8, h8, wia8, wib8,
                           bih8, whh8, bhh8, wf8, bf8)
        outs.append(o8.reshape(NP, FP)[:N, :NO])
    return jnp.stack(outs, axis=0)


# speedup vs baseline: 1.1326x; 1.1326x over previous
import functools

import jax
import jax.numpy as jnp
from jax import lax
from jax.experimental import pallas as pl
from jax.experimental.pallas import tpu as pltpu
from jax.experimental.pallas import tpu_sc as plsc

N = 10000
NP = 10240
NPB = NP // 8
E = 320000
ITERS = 7
NI = 9
NE = 11
NF = 10
NO = 9
MS = 96
FP = 16
G3 = 48

NC = 2
NS = 16
NW = NC * NS
EPW = E // NW
SW = 80
SPB = 5
BLK = SW * SPB
NBLK = EPW // BLK
IDXR = EPW // SW
RPT = NP // NS

EB = E // 8
WB = EPW // 8
BB = BLK // 8
RB = 800
MS8 = 8 * MS
_f32 = jnp.float32


def _mesh():
    return plsc.VectorSubcoreMesh(core_axis_name="c", subcore_axis_name="s",
                                  num_cores=NC, num_subcores=NS)


_SC_PARAMS = pltpu.CompilerParams(use_tc_tiling_on_sc=False)


def _gather_body(h_hbm, srcx, dstx, xs_hbm, xd_hbm,
                 idx_s, idx_d, rows_s, rows_d, stage, h_sh, sem_s, sem_d):
    c = lax.axis_index("c")
    s = lax.axis_index("s")
    wid = s * NC + c
    pltpu.sync_copy(h_hbm.at[pl.ds(s * RPT, RPT)], stage)
    pltpu.sync_copy(stage, h_sh.at[pl.ds(s * RPT, RPT)])
    pltpu.sync_copy(srcx.at[wid], idx_s)
    pltpu.sync_copy(dstx.at[wid], idx_d)
    plsc.subcore_barrier()

    def blk(b, carry):
        cps = []
        for t in range(SPB):
            r = b * SPB + t
            cps.append(pltpu.async_copy(h_sh.at[idx_s.at[r]],
                                        rows_s.at[pl.ds(t * SW, SW)], sem_s))
            cps.append(pltpu.async_copy(h_sh.at[idx_d.at[r]],
                                        rows_d.at[pl.ds(t * SW, SW)], sem_d))
        for cp in cps:
            cp.wait()
        off = wid * EPW + b * BLK
        pltpu.sync_copy(rows_s, xs_hbm.at[pl.ds(off, BLK)])
        pltpu.sync_copy(rows_d, xd_hbm.at[pl.ds(off, BLK)])
        return carry

    lax.fori_loop(0, NBLK, blk, 0)


def _gather_call(h, src2, dst2):
    out_type = (jax.ShapeDtypeStruct((E, FP), _f32),
                jax.ShapeDtypeStruct((E, FP), _f32))
    return pl.kernel(
        _gather_body,
        out_type=out_type,
        mesh=_mesh(),
        scratch_types=[
            pltpu.VMEM((IDXR, SW), jnp.int32),
            pltpu.VMEM((IDXR, SW), jnp.int32),
            pltpu.VMEM((BLK, FP), _f32),
            pltpu.VMEM((BLK, FP), _f32),
            pltpu.VMEM((RPT, FP), _f32),
            pltpu.VMEM_SHARED((NP, FP), _f32),
            pltpu.SemaphoreType.DMA,
            pltpu.SemaphoreType.DMA,
        ],
        compiler_params=_SC_PARAMS,
    )(h, src2, dst2)


def _scatter_body(msgs_hbm, dstx, aggp_hbm, idx_d, rows, flat, shared_agg, sem):
    c = lax.axis_index("c")
    s = lax.axis_index("s")
    wid = s * NC + c

    def zrow(i, carry):
        flat[i] = jnp.zeros((FP,), _f32)
        return carry

    lax.fori_loop(0, RPT, zrow, 0)
    pltpu.sync_copy(flat, shared_agg.at[pl.ds(s * RPT, RPT)])
    plsc.subcore_barrier()

    pltpu.sync_copy(dstx.at[wid], idx_d)

    def blk(b, carry):
        off = wid * EPW + b * BLK
        pltpu.sync_copy(msgs_hbm.at[pl.ds(off, BLK)], rows)
        cps = []
        for t in range(SPB):
            r = b * SPB + t
            cps.append(pltpu.async_copy(rows.at[pl.ds(t * SW, SW)],
                                        shared_agg.at[idx_d.at[r]], sem,
                                        add=True))
        for cp in cps:
            cp.wait()
        return carry

    lax.fori_loop(0, NBLK, blk, 0)
    plsc.subcore_barrier()
    pltpu.sync_copy(shared_agg.at[pl.ds(s * RPT, RPT)], flat)
    pltpu.sync_copy(flat, aggp_hbm.at[c, pl.ds(s * RPT, RPT)])


def _scatter_call(msgs, dst2):
    return pl.kernel(
        _scatter_body,
        out_type=jax.ShapeDtypeStruct((NC, NP, FP), _f32),
        mesh=_mesh(),
        scratch_types=[
            pltpu.VMEM((IDXR, SW), jnp.int32),
            pltpu.VMEM((BLK, FP), _f32),
            pltpu.VMEM((RPT, FP), _f32),
            pltpu.VMEM_SHARED((NP, FP), _f32),
            pltpu.SemaphoreType.DMA,
        ],
        compiler_params=_SC_PARAMS,
    )(msgs, dst2)


def _degree_body(dstx, degp_hbm, idx_d, ones, flat, shared_deg, sem):
    c = lax.axis_index("c")
    s = lax.axis_index("s")
    wid = s * NC + c

    def zrow(i, carry):
        flat[i] = jnp.zeros((FP,), _f32)
        return carry

    lax.fori_loop(0, RPT, zrow, 0)

    def orow(i, carry):
        ones[i] = jnp.ones((FP,), _f32)
        return carry

    lax.fori_loop(0, SW, orow, 0)
    pltpu.sync_copy(flat, shared_deg.at[pl.ds(s * RPT, RPT)])
    plsc.subcore_barrier()
    pltpu.sync_copy(dstx.at[wid], idx_d)

    def blk(b, carry):
        cps = []
        for t in range(SPB):
            r = b * SPB + t
            cps.append(pltpu.async_copy(ones, shared_deg.at[idx_d.at[r]], sem,
                                        add=True))
        for cp in cps:
            cp.wait()
        return carry

    lax.fori_loop(0, NBLK, blk, 0)
    plsc.subcore_barrier()
    pltpu.sync_copy(shared_deg.at[pl.ds(s * RPT, RPT)], flat)
    pltpu.sync_copy(flat, degp_hbm.at[c, pl.ds(s * RPT, RPT)])


def _degree_call(dst2):
    return pl.kernel(
        _degree_body,
        out_type=jax.ShapeDtypeStruct((NC, NP, FP), _f32),
        mesh=_mesh(),
        scratch_types=[
            pltpu.VMEM((IDXR, SW), jnp.int32),
            pltpu.VMEM((SW, FP), _f32),
            pltpu.VMEM((RPT, FP), _f32),
            pltpu.VMEM_SHARED((NP, FP), _f32),
            pltpu.SemaphoreType.DMA,
        ],
        compiler_params=_SC_PARAMS,
    )(dst2)


def _mlp_body(xs_ref, xd_ref, w1a_ref, w1b_ref, b1_ref, w2_ref, b2_ref,
              w3_ref, b3_ref, w4_ref, b4_ref, out_ref):
    h = (jnp.dot(xs_ref[...], w1a_ref[...], preferred_element_type=_f32)
         + jnp.dot(xd_ref[...], w1b_ref[...], preferred_element_type=_f32)
         + b1_ref[...])
    h = jnp.maximum(h, 0.0)
    h = jnp.dot(h, w2_ref[...], preferred_element_type=_f32) + b2_ref[...]
    h = jnp.maximum(h, 0.0)
    h = jnp.dot(h, w3_ref[...], preferred_element_type=_f32) + b3_ref[...]
    h = jnp.maximum(h, 0.0)
    out_ref[...] = (jnp.dot(h, w4_ref[...], preferred_element_type=_f32)
                    + b4_ref[...])


def _mlp_call(xs, xd, w1a, w1b, b1r, w2bd, b2r, w3bd, b3r, w4bd, b4r):
    def wspec(a):
        return pl.BlockSpec(a.shape, lambda i: (0,) * a.ndim)

    return pl.pallas_call(
        _mlp_body,
        grid=(EB // RB,),
        in_specs=[
            pl.BlockSpec((RB, 128), lambda i: (i, 0)),
            pl.BlockSpec((RB, 128), lambda i: (i, 0)),
            wspec(w1a), wspec(w1b), wspec(b1r), wspec(w2bd), wspec(b2r),
            wspec(w3bd), wspec(b3r), wspec(w4bd), wspec(b4r),
        ],
        out_specs=pl.BlockSpec((RB, 128), lambda i: (i, 0)),
        out_shape=jax.ShapeDtypeStruct((EB, 128), _f32),
    )(xs, xd, w1a, w1b, b1r, w2bd, b2r, w3bd, b3r, w4bd, b4r)


def _gru_body(aggp_ref, ni_ref, h_ref, wia_ref, wib_ref, bih_ref,
              whh_ref, bhh_ref, wf_ref, bf_ref, hout_ref, o_ref):
    agg = aggp_ref[0] + aggp_ref[1]
    gi = (jnp.dot(agg, wia_ref[...], preferred_element_type=_f32)
          + jnp.dot(ni_ref[...], wib_ref[...], preferred_element_type=_f32)
          + bih_ref[...])
    h = h_ref[...]
    gh = jnp.dot(h, whh_ref[...], preferred_element_type=_f32) + bhh_ref[...]
    r = jax.nn.sigmoid(gi[:, 0:128] + gh[:, 0:128])
    z = jax.nn.sigmoid(gi[:, 128:256] + gh[:, 128:256])
    n = jnp.tanh(gi[:, 256:384] + r * gh[:, 256:384])
    hn = (1.0 - z) * n + z * h
    hout_ref[...] = hn
    o_ref[...] = jnp.dot(hn, wf_ref[...], preferred_element_type=_f32) + bf_ref[...]


def _gru_call(aggp8, ni8, h8, wia8, wib8, bih8, whh8, bhh8, wf8, bf8):
    return pl.pallas_call(
        _gru_body,
        out_shape=(jax.ShapeDtypeStruct((NPB, 128), _f32),
                   jax.ShapeDtypeStruct((NPB, 128), _f32)),
    )(aggp8, ni8, h8, wia8, wib8, bih8, whh8, bhh8, wf8, bf8)


def _gru0_body(degp_ref, ni_ref, wia_ref, wib_ref, bih_ref, bhh_ref,
               wf_ref, bf_ref, b1_ref, w2_ref, b2_ref, w3_ref, b3_ref,
               w4_ref, b4_ref, hout_ref, o_ref):
    t = jnp.maximum(b1_ref[...], 0.0)
    t = jnp.maximum(
        jnp.dot(t, w2_ref[...], preferred_element_type=_f32) + b2_ref[...], 0.0)
    t = jnp.maximum(
        jnp.dot(t, w3_ref[...], preferred_element_type=_f32) + b3_ref[...], 0.0)
    m0 = jnp.dot(t, w4_ref[...], preferred_element_type=_f32) + b4_ref[...]
    agg = (degp_ref[0] + degp_ref[1]) * jnp.tile(m0, (1, 8))
    gi = (jnp.dot(agg, wia_ref[...], preferred_element_type=_f32)
          + jnp.dot(ni_ref[...], wib_ref[...], preferred_element_type=_f32)
          + bih_ref[...])
    gh = bhh_ref[...]
    r = jax.nn.sigmoid(gi[:, 0:128] + gh[:, 0:128])
    z = jax.nn.sigmoid(gi[:, 128:256] + gh[:, 128:256])
    n = jnp.tanh(gi[:, 256:384] + r * gh[:, 256:384])
    hn = (1.0 - z) * n
    hout_ref[...] = hn
    o_ref[...] = jnp.dot(hn, wf_ref[...], preferred_element_type=_f32) + bf_ref[...]


def _gru0_call(degp8, ni8, wia8, wib8, bih8, bhh8, wf8, bf8,
               b1o, w2o, b2o, w3o, b3o, w4o, b4o):
    return pl.pallas_call(
        _gru0_body,
        out_shape=(jax.ShapeDtypeStruct((NPB, 128), _f32),
                   jax.ShapeDtypeStruct((NPB, 128), _f32)),
    )(degp8, ni8, wia8, wib8, bih8, bhh8, wf8, bf8,
      b1o, w2o, b2o, w3o, b3o, w4o, b4o)


def kernel(node_inputs, src_ids, dst_ids, W1, b1, W2, b2, W3, b3, W4, b4,
           W_ih, b_ih, W_hh, b_hh, Wf, bf):
    src2 = src_ids.astype(jnp.int32).reshape(NW, IDXR, SW)
    dst2 = dst_ids.astype(jnp.int32).reshape(NW, IDXR, SW)
    ni_p = jnp.pad(node_inputs.astype(_f32), ((0, NP - N), (0, FP - NI)))

    eye8 = jnp.eye(8, dtype=_f32)
    w1a = jnp.kron(eye8, jnp.pad(W1[0:NF], ((0, FP - NF), (0, 0))))
    w1b = jnp.kron(eye8, jnp.pad(W1[NF:2 * NF], ((0, FP - NF), (0, 0))))
    w2bd = jnp.kron(eye8, W2)
    w3bd = jnp.kron(eye8, W3)
    w4bd = jnp.kron(eye8, jnp.pad(W4, ((0, 0), (0, FP - NE))))
    b4r = jnp.tile(jnp.pad(b4, (0, FP - NE)), 8)[None, :]
    b1r = jnp.tile(b1, 8)[None, :]
    b2r = jnp.tile(b2, 8)[None, :]
    b3r = jnp.tile(b3, 8)[None, :]

    def gate_pack(w, nrow):
        wp = jnp.pad(w.reshape(nrow, 3, NF),
                     ((0, FP - nrow), (0, 0), (0, FP - NF)))
        return jnp.concatenate(
            [jnp.kron(eye8, wp[:, g, :]) for g in range(3)], axis=1)

    wia8 = gate_pack(W_ih[0:NE], NE)
    wib8 = gate_pack(W_ih[NE:NE + NI], NI)
    whh8 = gate_pack(W_hh, NF)
    bp_ih = jnp.pad(b_ih.reshape(3, NF), ((0, 0), (0, FP - NF)))
    bih8 = jnp.tile(bp_ih, (1, 8)).reshape(1, 384)
    bp_hh = jnp.pad(b_hh.reshape(3, NF), ((0, 0), (0, FP - NF)))
    bhh8 = jnp.tile(bp_hh, (1, 8)).reshape(1, 384)
    wf8 = jnp.kron(eye8, jnp.pad(Wf, ((0, FP - NF), (0, FP - NO))))
    bf8 = jnp.tile(jnp.pad(bf, (0, FP - NO)), 8)[None, :]
    ni8 = ni_p.reshape(NPB, 128)

    b1o = b1[None, :]
    b2o = b2[None, :]
    b3o = b3[None, :]
    w4o = jnp.zeros((MS, FP), _f32).at[:, :NE].set(W4)
    b4o = jnp.zeros((1, FP), _f32).at[0, :NE].set(b4)

    outs = []
    degp = _degree_call(dst2)
    h8, o8 = _gru0_call(degp.reshape(NC, NPB, 128), ni8, wia8, wib8, bih8,
                        bhh8, wf8, bf8, b1o, W2, b2o, W3, b3o, w4o, b4o)
    outs.append(o8.reshape(NP, FP)[:N, :NO])
    for _ in range(ITERS - 1):
        xs, xd = _gather_call(h8.reshape(NP, FP), src2, dst2)
        msgs8 = _mlp_call(xs.reshape(EB, 128), xd.reshape(EB, 128),
                          w1a, w1b, b1r, w2bd, b2r, w3bd, b3r, w4bd, b4r)
        aggp = _scatter_call(msgs8.reshape(E, FP), dst2)
        h8, o8 = _gru_call(aggp.reshape(NC, NPB, 128), ni8, h8, wia8, wib8,
                           bih8, whh8, bhh8, wf8, bf8)
        outs.append(o8.reshape(NP, FP)[:N, :NO])
    return jnp.stack(outs, axis=0)


# speedup vs baseline: 1.1329x; 1.0003x over previous
import functools

import jax
import jax.numpy as jnp
from jax import lax
from jax.experimental import pallas as pl
from jax.experimental.pallas import tpu as pltpu
from jax.experimental.pallas import tpu_sc as plsc

N = 10000
NP = 10240
NPB = NP // 8
E = 320000
ITERS = 7
NI = 9
NE = 11
NF = 10
NO = 9
MS = 96
FP = 16

NC = 2
NS = 16
NW = NC * NS
EPW = E // NW
SW = 80
SPB = 5
BLK = SW * SPB
NBLK = EPW // BLK
IDXR = EPW // SW
RPT = NP // NS

EB = E // 8
RB = 800
_f32 = jnp.float32


def _mesh():
    return plsc.VectorSubcoreMesh(core_axis_name="c", subcore_axis_name="s",
                                  num_cores=NC, num_subcores=NS)


_SC_PARAMS = pltpu.CompilerParams(use_tc_tiling_on_sc=False)


def _gather_body(h_hbm, srcx, dstx, xs_hbm, xd_hbm,
                 idx_s, idx_d, rows_s, rows_d, stage, h_sh, sem_s, sem_d):
    c = lax.axis_index("c")
    s = lax.axis_index("s")
    wid = s * NC + c
    pltpu.sync_copy(h_hbm.at[pl.ds(s * RPT, RPT)], stage)
    pltpu.sync_copy(stage, h_sh.at[pl.ds(s * RPT, RPT)])
    pltpu.sync_copy(srcx.at[wid], idx_s)
    pltpu.sync_copy(dstx.at[wid], idx_d)
    plsc.subcore_barrier()

    def blk(b, carry):
        cps = []
        for t in range(SPB):
            r = b * SPB + t
            cps.append(pltpu.async_copy(h_sh.at[idx_s.at[r]],
                                        rows_s.at[pl.ds(t * SW, SW)], sem_s))
            cps.append(pltpu.async_copy(h_sh.at[idx_d.at[r]],
                                        rows_d.at[pl.ds(t * SW, SW)], sem_d))
        for cp in cps:
            cp.wait()
        off = wid * EPW + b * BLK
        pltpu.sync_copy(rows_s, xs_hbm.at[pl.ds(off, BLK)])
        pltpu.sync_copy(rows_d, xd_hbm.at[pl.ds(off, BLK)])
        return carry

    lax.fori_loop(0, NBLK, blk, 0)


def _gather_call(h, src2, dst2):
    out_type = (jax.ShapeDtypeStruct((E, FP), _f32),
                jax.ShapeDtypeStruct((E, FP), _f32))
    return pl.kernel(
        _gather_body,
        out_type=out_type,
        mesh=_mesh(),
        scratch_types=[
            pltpu.VMEM((IDXR, SW), jnp.int32),
            pltpu.VMEM((IDXR, SW), jnp.int32),
            pltpu.VMEM((BLK, FP), _f32),
            pltpu.VMEM((BLK, FP), _f32),
            pltpu.VMEM((RPT, FP), _f32),
            pltpu.VMEM_SHARED((NP, FP), _f32),
            pltpu.SemaphoreType.DMA,
            pltpu.SemaphoreType.DMA,
        ],
        compiler_params=_SC_PARAMS,
    )(h, src2, dst2)


def _scatter_body(msgs_hbm, dstx, aggp_hbm, idx_d, rows, flat, shared_agg, sem):
    c = lax.axis_index("c")
    s = lax.axis_index("s")
    wid = s * NC + c

    def zrow(i, carry):
        flat[i] = jnp.zeros((FP,), _f32)
        return carry

    lax.fori_loop(0, RPT, zrow, 0)
    pltpu.sync_copy(flat, shared_agg.at[pl.ds(s * RPT, RPT)])
    plsc.subcore_barrier()

    pltpu.sync_copy(dstx.at[wid], idx_d)

    def blk(b, carry):
        off = wid * EPW + b * BLK
        pltpu.sync_copy(msgs_hbm.at[pl.ds(off, BLK)], rows)
        cps = []
        for t in range(SPB):
            r = b * SPB + t
            cps.append(pltpu.async_copy(rows.at[pl.ds(t * SW, SW)],
                                        shared_agg.at[idx_d.at[r]], sem,
                                        add=True))
        for cp in cps:
            cp.wait()
        return carry

    lax.fori_loop(0, NBLK, blk, 0)
    plsc.subcore_barrier()
    pltpu.sync_copy(shared_agg.at[pl.ds(s * RPT, RPT)], flat)
    pltpu.sync_copy(flat, aggp_hbm.at[c, pl.ds(s * RPT, RPT)])


def _scatter_call(msgs, dst2):
    return pl.kernel(
        _scatter_body,
        out_type=jax.ShapeDtypeStruct((NC, NP, FP), _f32),
        mesh=_mesh(),
        scratch_types=[
            pltpu.VMEM((IDXR, SW), jnp.int32),
            pltpu.VMEM((BLK, FP), _f32),
            pltpu.VMEM((RPT, FP), _f32),
            pltpu.VMEM_SHARED((NP, FP), _f32),
            pltpu.SemaphoreType.DMA,
        ],
        compiler_params=_SC_PARAMS,
    )(msgs, dst2)


def _degree_body(dstx, degp_hbm, idx_d, ones, flat, shared_deg, sem):
    c = lax.axis_index("c")
    s = lax.axis_index("s")
    wid = s * NC + c

    def zrow(i, carry):
        flat[i] = jnp.zeros((FP,), _f32)
        return carry

    lax.fori_loop(0, RPT, zrow, 0)

    def orow(i, carry):
        ones[i] = jnp.ones((FP,), _f32)
        return carry

    lax.fori_loop(0, SW, orow, 0)
    pltpu.sync_copy(flat, shared_deg.at[pl.ds(s * RPT, RPT)])
    plsc.subcore_barrier()
    pltpu.sync_copy(dstx.at[wid], idx_d)

    def blk(b, carry):
        cps = []
        for t in range(SPB):
            r = b * SPB + t
            cps.append(pltpu.async_copy(ones, shared_deg.at[idx_d.at[r]], sem,
                                        add=True))
        for cp in cps:
            cp.wait()
        return carry

    lax.fori_loop(0, NBLK, blk, 0)
    plsc.subcore_barrier()
    pltpu.sync_copy(shared_deg.at[pl.ds(s * RPT, RPT)], flat)
    pltpu.sync_copy(flat, degp_hbm.at[c, pl.ds(s * RPT, RPT)])


def _degree_call(dst2):
    return pl.kernel(
        _degree_body,
        out_type=jax.ShapeDtypeStruct((NC, NP, FP), _f32),
        mesh=_mesh(),
        scratch_types=[
            pltpu.VMEM((IDXR, SW), jnp.int32),
            pltpu.VMEM((SW, FP), _f32),
            pltpu.VMEM((RPT, FP), _f32),
            pltpu.VMEM_SHARED((NP, FP), _f32),
            pltpu.SemaphoreType.DMA,
        ],
        compiler_params=_SC_PARAMS,
    )(dst2)


def _mlp_body(xs_ref, xd_ref, w1a_ref, w1b_ref, b1_ref, w2_ref, b2_ref,
              w3_ref, b3_ref, w4_ref, b4_ref, out_ref):
    h = (jnp.dot(xs_ref[...], w1a_ref[...], preferred_element_type=_f32)
         + jnp.dot(xd_ref[...], w1b_ref[...], preferred_element_type=_f32)
         + b1_ref[...])
    h = jnp.maximum(h, 0.0)
    h = jnp.dot(h, w2_ref[...], preferred_element_type=_f32) + b2_ref[...]
    h = jnp.maximum(h, 0.0)
    h = jnp.dot(h, w3_ref[...], preferred_element_type=_f32) + b3_ref[...]
    h = jnp.maximum(h, 0.0)
    out_ref[...] = (jnp.dot(h, w4_ref[...], preferred_element_type=_f32)
                    + b4_ref[...])


def _mlp_call(xs, xd, w1a, w1b, b1r, w2bd, b2r, w3bd, b3r, w4bd, b4r):
    def wspec(a):
        return pl.BlockSpec(a.shape, lambda i: (0,) * a.ndim)

    return pl.pallas_call(
        _mlp_body,
        grid=(EB // RB,),
        in_specs=[
            pl.BlockSpec((RB, 128), lambda i: (i, 0)),
            pl.BlockSpec((RB, 128), lambda i: (i, 0)),
            wspec(w1a), wspec(w1b), wspec(b1r), wspec(w2bd), wspec(b2r),
            wspec(w3bd), wspec(b3r), wspec(w4bd), wspec(b4r),
        ],
        out_specs=pl.BlockSpec((RB, 128), lambda i: (i, 0)),
        out_shape=jax.ShapeDtypeStruct((EB, 128), _f32),
    )(xs, xd, w1a, w1b, b1r, w2bd, b2r, w3bd, b3r, w4bd, b4r)


def _gru_body(aggp_ref, ni_ref, h_ref, wia_ref, wib_ref, bih_ref,
              whh_ref, bhh_ref, wf_ref, bf_ref, hout_ref, o_ref):
    agg = aggp_ref[0] + aggp_ref[1]
    gi = (jnp.dot(agg, wia_ref[...], preferred_element_type=_f32)
          + jnp.dot(ni_ref[...], wib_ref[...], preferred_element_type=_f32)
          + bih_ref[...])
    h = h_ref[...]
    gh = jnp.dot(h, whh_ref[...], preferred_element_type=_f32) + bhh_ref[...]
    r = jax.nn.sigmoid(gi[:, 0:128] + gh[:, 0:128])
    z = jax.nn.sigmoid(gi[:, 128:256] + gh[:, 128:256])
    n = jnp.tanh(gi[:, 256:384] + r * gh[:, 256:384])
    hn = (1.0 - z) * n + z * h
    hout_ref[...] = hn
    o_ref[...] = jnp.dot(hn, wf_ref[...], preferred_element_type=_f32) + bf_ref[...]


def _gru_call(aggp8, ni8, h8, wia8, wib8, bih8, whh8, bhh8, wf8, bf8):
    return pl.pallas_call(
        _gru_body,
        out_shape=(jax.ShapeDtypeStruct((NPB, 128), _f32),
                   jax.ShapeDtypeStruct((NPB, 128), _f32)),
    )(aggp8, ni8, h8, wia8, wib8, bih8, whh8, bhh8, wf8, bf8)


def _gru0_body(degp_ref, ni_ref, wia_ref, wib_ref, bih_ref, bhh_ref,
               wf_ref, bf_ref, b1_ref, w2_ref, b2_ref, w3_ref, b3_ref,
               w4_ref, b4_ref, hout_ref, o_ref):
    t = jnp.maximum(b1_ref[...], 0.0)
    t = jnp.maximum(
        jnp.dot(t, w2_ref[...], preferred_element_type=_f32) + b2_ref[...], 0.0)
    t = jnp.maximum(
        jnp.dot(t, w3_ref[...], preferred_element_type=_f32) + b3_ref[...], 0.0)
    m0 = jnp.dot(t, w4_ref[...], preferred_element_type=_f32) + b4_ref[...]
    agg = (degp_ref[0] + degp_ref[1]) * jnp.tile(m0, (1, 8))
    gi = (jnp.dot(agg, wia_ref[...], preferred_element_type=_f32)
          + jnp.dot(ni_ref[...], wib_ref[...], preferred_element_type=_f32)
          + bih_ref[...])
    gh = bhh_ref[...]
    r = jax.nn.sigmoid(gi[:, 0:128] + gh[:, 0:128])
    z = jax.nn.sigmoid(gi[:, 128:256] + gh[:, 128:256])
    n = jnp.tanh(gi[:, 256:384] + r * gh[:, 256:384])
    hn = (1.0 - z) * n
    hout_ref[...] = hn
    o_ref[...] = jnp.dot(hn, wf_ref[...], preferred_element_type=_f32) + bf_ref[...]


def _gru0_call(degp8, ni8, wia8, wib8, bih8, bhh8, wf8, bf8,
               b1o, w2o, b2o, w3o, b3o, w4o, b4o):
    return pl.pallas_call(
        _gru0_body,
        out_shape=(jax.ShapeDtypeStruct((NPB, 128), _f32),
                   jax.ShapeDtypeStruct((NPB, 128), _f32)),
    )(degp8, ni8, wia8, wib8, bih8, bhh8, wf8, bf8,
      b1o, w2o, b2o, w3o, b3o, w4o, b4o)


def kernel(node_inputs, src_ids, dst_ids, W1, b1, W2, b2, W3, b3, W4, b4,
           W_ih, b_ih, W_hh, b_hh, Wf, bf):
    src2 = src_ids.astype(jnp.int32).reshape(NW, IDXR, SW)
    dst2 = dst_ids.astype(jnp.int32).reshape(NW, IDXR, SW)
    ni_p = jnp.pad(node_inputs.astype(_f32), ((0, NP - N), (0, FP - NI)))

    eye8 = jnp.eye(8, dtype=_f32)
    w1a = jnp.kron(eye8, jnp.pad(W1[0:NF], ((0, FP - NF), (0, 0))))
    w1b = jnp.kron(eye8, jnp.pad(W1[NF:2 * NF], ((0, FP - NF), (0, 0))))
    w2bd = jnp.kron(eye8, W2)
    w3bd = jnp.kron(eye8, W3)
    w4bd = jnp.kron(eye8, jnp.pad(W4, ((0, 0), (0, FP - NE))))
    b4r = jnp.tile(jnp.pad(b4, (0, FP - NE)), 8)[None, :]
    b1r = jnp.tile(b1, 8)[None, :]
    b2r = jnp.tile(b2, 8)[None, :]
    b3r = jnp.tile(b3, 8)[None, :]

    def gate_pack(w, nrow):
        wp = jnp.pad(w.reshape(nrow, 3, NF),
                     ((0, FP - nrow), (0, 0), (0, FP - NF)))
        return jnp.concatenate(
            [jnp.kron(eye8, wp[:, g, :]) for g in range(3)], axis=1)

    wia8 = gate_pack(W_ih[0:NE], NE)
    wib8 = gate_pack(W_ih[NE:NE + NI], NI)
    whh8 = gate_pack(W_hh, NF)
    bp_ih = jnp.pad(b_ih.reshape(3, NF), ((0, 0), (0, FP - NF)))
    bih8 = jnp.tile(bp_ih, (1, 8)).reshape(1, 384)
    bp_hh = jnp.pad(b_hh.reshape(3, NF), ((0, 0), (0, FP - NF)))
    bhh8 = jnp.tile(bp_hh, (1, 8)).reshape(1, 384)
    wf8 = jnp.kron(eye8, jnp.pad(Wf, ((0, FP - NF), (0, FP - NO))))
    bf8 = jnp.tile(jnp.pad(bf, (0, FP - NO)), 8)[None, :]
    ni8 = ni_p.reshape(NPB, 128)

    b1o = b1[None, :]
    b2o = b2[None, :]
    b3o = b3[None, :]
    w4o = jnp.zeros((MS, FP), _f32).at[:, :NE].set(W4)
    b4o = jnp.zeros((1, FP), _f32).at[0, :NE].set(b4)

    outs = []
    degp = _degree_call(dst2)
    h8, o8 = _gru0_call(degp.reshape(NC, NPB, 128), ni8, wia8, wib8, bih8,
                        bhh8, wf8, bf8, b1o, W2, b2o, W3, b3o, w4o, b4o)
    outs.append(o8.reshape(NP, FP)[:N, :NO])
    for _ in range(ITERS - 1):
        xs, xd = _gather_call(h8.reshape(NP, FP), src2, dst2)
        msgs8 = _mlp_call(xs.reshape(EB, 128), xd.reshape(EB, 128),
                          w1a, w1b, b1r, w2bd, b2r, w3bd, b3r, w4bd, b4r)
        aggp = _scatter_call(msgs8.reshape(E, FP), dst2)
        h8, o8 = _gru_call(aggp.reshape(NC, NPB, 128), ni8, h8, wia8, wib8,
                           bih8, whh8, bhh8, wf8, bf8)
        outs.append(o8.reshape(NP, FP)[:N, :NO])
    return jnp.stack(outs, axis=0)
